# Initial kernel scaffold; baseline (speedup 1.0000x reference)
import jax
import jax.numpy as jnp
from jax.experimental import pallas as pl


def kernel(x, router_w, w1, w2):
    raise NotImplementedError("write your pallas kernel here")


# speedup vs baseline: 1.4871x; 1.4871x over previous
import functools

import jax
import jax.numpy as jnp
from jax.experimental import pallas as pl
from jax.experimental.pallas import tpu as pltpu

_N_EMBD = 1024
_NUM_EXPERTS = 64
_EXPERT_WIDTH = 128
_TOP_K = 8
_E_BLK = 4


def _router_gate_kernel(x_ref, rw_ref, gate_ref):
    x = x_ref[...]
    logits = jax.lax.dot_general(
        x, rw_ref[...], (((1,), (1,)), ((), ())),
        preferred_element_type=jnp.float32)
    m = jnp.max(logits, axis=1, keepdims=True)
    p = jnp.exp(logits - m)
    p = p / jnp.sum(p, axis=1, keepdims=True)
    t, e = p.shape
    col = jax.lax.broadcasted_iota(jnp.int32, (t, e), 1)
    gate = jnp.zeros((t, e), jnp.float32)
    tot = jnp.zeros((t, 1), jnp.float32)
    for _ in range(_TOP_K):
        mv = jnp.max(p, axis=1, keepdims=True)
        sel = jnp.min(jnp.where(p == mv, col, e), axis=1, keepdims=True)
        chosen = col == sel
        gate = gate + jnp.where(chosen, mv, 0.0)
        tot = tot + mv
        p = jnp.where(chosen, -1.0, p)
    gate_ref[...] = gate / tot


def _mlp_kernel(xbf_ref, gate_ref, w1_ref, w2_ref, out_ref):
    eb = pl.program_id(0)
    x = xbf_ref[...]
    w1 = w1_ref[...].astype(jnp.bfloat16)
    h = jnp.dot(x, w1, preferred_element_type=jnp.float32)
    a = jax.nn.gelu(h)
    gate = gate_ref[0]
    parts = [
        a[:, e * _EXPERT_WIDTH:(e + 1) * _EXPERT_WIDTH] * gate[:, e:e + 1]
        for e in range(_E_BLK)
    ]
    hg = jnp.concatenate(parts, axis=1).astype(jnp.bfloat16)
    w2 = w2_ref[...].astype(jnp.bfloat16)
    part = jnp.dot(hg, w2, preferred_element_type=jnp.float32)

    @pl.when(eb == 0)
    def _():
        out_ref[...] = part

    @pl.when(eb != 0)
    def _():
        out_ref[...] += part


def kernel(x, router_w, w1, w2):
    b, s, d = x.shape
    t = b * s
    xt = x.reshape(t, d)
    xbf = xt.astype(jnp.bfloat16)

    gate = pl.pallas_call(
        _router_gate_kernel,
        out_shape=jax.ShapeDtypeStruct((t, _NUM_EXPERTS), jnp.float32),
    )(xt, router_w)

    neb = _NUM_EXPERTS // _E_BLK
    bw = _E_BLK * _EXPERT_WIDTH
    gate3 = gate.reshape(t, neb, _E_BLK).transpose(1, 0, 2)
    out = pl.pallas_call(
        _mlp_kernel,
        grid=(neb,),
        in_specs=[
            pl.BlockSpec((t, d), lambda i: (0, 0)),
            pl.BlockSpec((1, t, _E_BLK), lambda i: (i, 0, 0)),
            pl.BlockSpec((d, bw), lambda i: (0, i)),
            pl.BlockSpec((bw, d), lambda i: (i, 0)),
        ],
        out_specs=pl.BlockSpec((t, d), lambda i: (0, 0)),
        out_shape=jax.ShapeDtypeStruct((t, d), jnp.float32),
        compiler_params=pltpu.CompilerParams(
            dimension_semantics=("arbitrary",)),
    )(xbf, gate3, w1, w2)
    return out.reshape(b, s, d)


# speedup vs baseline: 1.6104x; 1.0829x over previous
import functools

import jax
import jax.numpy as jnp
from jax.experimental import pallas as pl
from jax.experimental.pallas import tpu as pltpu

_N_EMBD = 1024
_NUM_EXPERTS = 64
_EXPERT_WIDTH = 128
_TOP_K = 8
_E_BLK = 8


def _router_gate_kernel(x_ref, rw_ref, gate_ref):
    x = x_ref[...]
    logits = jax.lax.dot_general(
        x, rw_ref[...], (((1,), (1,)), ((), ())),
        preferred_element_type=jnp.float32)
    m = jnp.max(logits, axis=1, keepdims=True)
    p = jnp.exp(logits - m)
    p = p / jnp.sum(p, axis=1, keepdims=True)
    t, e = p.shape
    col = jax.lax.broadcasted_iota(jnp.int32, (t, e), 1)
    gate = jnp.zeros((t, e), jnp.float32)
    tot = jnp.zeros((t, 1), jnp.float32)
    for _ in range(_TOP_K):
        mv = jnp.max(p, axis=1, keepdims=True)
        sel = jnp.min(jnp.where(p == mv, col, e), axis=1, keepdims=True)
        chosen = col == sel
        gate = gate + jnp.where(chosen, mv, 0.0)
        tot = tot + mv
        p = jnp.where(chosen, -1.0, p)
    gate_ref[...] = gate / tot


_SQRT_2_OVER_PI = 0.7978845608028654
_GELU_C = 0.044715


def _mlp_kernel(xbf_ref, gate_ref, w1_ref, w2_ref, out_ref, hg_ref):
    eb = pl.program_id(0)
    x = xbf_ref[...]
    w1 = w1_ref[...].astype(jnp.bfloat16)
    h = jnp.dot(x, w1, preferred_element_type=jnp.float32)
    gate = gate_ref[0]
    f = _EXPERT_WIDTH
    for e in range(_E_BLK):
        he = h[:, e * f:(e + 1) * f]
        ge = gate[:, e:e + 1] * 0.5
        u = (_SQRT_2_OVER_PI * he) * (1.0 + _GELU_C * (he * he))
        hg = (ge * he) * (1.0 + jnp.tanh(u))
        hg_ref[:, e * f:(e + 1) * f] = hg.astype(jnp.bfloat16)
    w2 = w2_ref[...].astype(jnp.bfloat16)
    part = jnp.dot(hg_ref[...], w2, preferred_element_type=jnp.float32)

    @pl.when(eb == 0)
    def _():
        out_ref[...] = part

    @pl.when(eb != 0)
    def _():
        out_ref[...] += part


def kernel(x, router_w, w1, w2):
    b, s, d = x.shape
    t = b * s
    xt = x.reshape(t, d)
    xbf = xt.astype(jnp.bfloat16)

    gate = pl.pallas_call(
        _router_gate_kernel,
        out_shape=jax.ShapeDtypeStruct((t, _NUM_EXPERTS), jnp.float32),
    )(xt, router_w)

    neb = _NUM_EXPERTS // _E_BLK
    bw = _E_BLK * _EXPERT_WIDTH
    gate3 = gate.reshape(t, neb, _E_BLK).transpose(1, 0, 2)
    out = pl.pallas_call(
        _mlp_kernel,
        grid=(neb,),
        in_specs=[
            pl.BlockSpec((t, d), lambda i: (0, 0)),
            pl.BlockSpec((1, t, _E_BLK), lambda i: (i, 0, 0)),
            pl.BlockSpec((d, bw), lambda i: (0, i)),
            pl.BlockSpec((bw, d), lambda i: (i, 0)),
        ],
        out_specs=pl.BlockSpec((t, d), lambda i: (0, 0)),
        out_shape=jax.ShapeDtypeStruct((t, d), jnp.float32),
        scratch_shapes=[pltpu.VMEM((t, bw), jnp.bfloat16)],
        compiler_params=pltpu.CompilerParams(
            dimension_semantics=("arbitrary",)),
    )(xbf, gate3, w1, w2)
    return out.reshape(b, s, d)
